# Initial kernel scaffold; baseline (speedup 1.0000x reference)
import jax
import jax.numpy as jnp
from jax.experimental import pallas as pl


def kernel(x, edge_index, edge_attr, batch, Ws, bias):
    raise NotImplementedError("write your pallas kernel here")


# speedup vs baseline: 8.2204x; 8.2204x over previous
import functools
import jax
import jax.numpy as jnp
from jax import lax
from jax.experimental import pallas as pl
from jax.experimental.pallas import tpu as pltpu
from jax.experimental.pallas import tpu_sc as plsc

N = 10000
D = 128
KHOPS = 3
NC, NS = 2, 16
NW = NC * NS
CHUNK = 128
NA = 10112
SLAB = NA // NS
ROWBLK = 1000


def _mesh():
    return plsc.VectorSubcoreMesh(core_axis_name="c", subcore_axis_name="s")


def _make_sc_degree(n_chunks):
    @functools.partial(
        pl.kernel,
        out_type=jax.ShapeDtypeStruct((NC, NA, D), jnp.float32),
        mesh=_mesh(),
        scratch_types=[
            pltpu.VMEM((n_chunks, CHUNK), jnp.int32),
            pltpu.VMEM((CHUNK, D), jnp.float32),
            pltpu.VMEM_SHARED((NA, D), jnp.float32),
        ],
    )
    def deg_kernel(col_hbm, ones_hbm, zeros_hbm, out_hbm, cvec, ones_v, acc):
        cid = lax.axis_index("c")
        sid = lax.axis_index("s")
        wid = cid * NS + sid
        pltpu.sync_copy(col_hbm.at[wid], cvec)
        pltpu.sync_copy(ones_hbm, ones_v)
        r0 = sid * SLAB
        pltpu.sync_copy(zeros_hbm.at[pl.ds(r0, SLAB)], acc.at[pl.ds(r0, SLAB)])
        plsc.subcore_barrier()

        def body(j, carry):
            pltpu.sync_copy(ones_v, acc.at[cvec.at[j]], add=True)
            return carry

        lax.fori_loop(0, n_chunks, body, 0)
        plsc.subcore_barrier()
        pltpu.sync_copy(acc.at[pl.ds(r0, SLAB)], out_hbm.at[cid, pl.ds(r0, SLAB)])

    return deg_kernel


def _make_sc_hop(n_chunks):
    @functools.partial(
        pl.kernel,
        out_type=jax.ShapeDtypeStruct((NC, NA, D), jnp.float32),
        mesh=_mesh(),
        scratch_types=[
            pltpu.VMEM((n_chunks, CHUNK), jnp.int32),
            pltpu.VMEM((n_chunks, CHUNK), jnp.int32),
            pltpu.VMEM((CHUNK, D), jnp.float32),
            pltpu.VMEM_SHARED((NA, D), jnp.float32),
            pltpu.SemaphoreType.DMA,
        ],
    )
    def hop_kernel(g_hbm, row_hbm, col_hbm, zeros_hbm, out_hbm,
                   rvec, cvec, rows_v, acc, sem):
        cid = lax.axis_index("c")
        sid = lax.axis_index("s")
        wid = cid * NS + sid
        pltpu.sync_copy(row_hbm.at[wid], rvec)
        pltpu.sync_copy(col_hbm.at[wid], cvec)
        r0 = sid * SLAB
        pltpu.sync_copy(zeros_hbm.at[pl.ds(r0, SLAB)], acc.at[pl.ds(r0, SLAB)])
        plsc.subcore_barrier()

        def body(j, carry):
            pltpu.async_copy(g_hbm.at[rvec.at[j]], rows_v, sem).wait()
            pltpu.sync_copy(rows_v, acc.at[cvec.at[j]], add=True)
            return carry

        lax.fori_loop(0, n_chunks, body, 0)
        plsc.subcore_barrier()
        pltpu.sync_copy(acc.at[pl.ds(r0, SLAB)], out_hbm.at[cid, pl.ds(r0, SLAB)])

    return hop_kernel


def _rowspec():
    return pl.BlockSpec((ROWBLK, D), lambda i: (i, 0))


def _wspec():
    return pl.BlockSpec((D, D), lambda i: (0, 0))


def _tc_prep_body(x_ref, d0_ref, d1_ref, w_ref, g_ref, acc_ref, dinv_ref):
    deg = d0_ref[:, :1] + d1_ref[:, :1]
    dinv = jnp.where(deg > 0, lax.rsqrt(deg), 0.0)
    dinv_b = jnp.broadcast_to(dinv, (ROWBLK, D))
    dinv_ref[...] = dinv_b
    x = x_ref[...]
    g_ref[...] = x * dinv_b
    acc_ref[...] = jnp.dot(x, w_ref[...], preferred_element_type=jnp.float32)


def _tc_prep(x, d0, d1, w0):
    return pl.pallas_call(
        _tc_prep_body,
        grid=(N // ROWBLK,),
        in_specs=[_rowspec(), _rowspec(), _rowspec(), _wspec()],
        out_specs=[_rowspec(), _rowspec(), _rowspec()],
        out_shape=[jax.ShapeDtypeStruct((N, D), jnp.float32)] * 3,
    )(x, d0, d1, w0)


def _tc_hop_body(s0_ref, s1_ref, dinv_ref, w_ref, accin_ref, g_ref, acc_ref):
    dinv = dinv_ref[...]
    h = (s0_ref[...] + s1_ref[...]) * dinv
    g_ref[...] = h * dinv
    acc_ref[...] = accin_ref[...] + jnp.dot(
        h, w_ref[...], preferred_element_type=jnp.float32)


def _tc_hop(s0, s1, dinv, wk, acc):
    return pl.pallas_call(
        _tc_hop_body,
        grid=(N // ROWBLK,),
        in_specs=[_rowspec(), _rowspec(), _rowspec(), _wspec(), _rowspec()],
        out_specs=[_rowspec(), _rowspec()],
        out_shape=[jax.ShapeDtypeStruct((N, D), jnp.float32)] * 2,
    )(s0, s1, dinv, wk, acc)


def _tc_final_body(s0_ref, s1_ref, dinv_ref, w_ref, accin_ref, b_ref, o_ref):
    h = (s0_ref[...] + s1_ref[...]) * dinv_ref[...]
    o = accin_ref[...] + jnp.dot(h, w_ref[...], preferred_element_type=jnp.float32)
    o_ref[...] = jnp.maximum(o + b_ref[...], 0.0)


def _tc_final(s0, s1, dinv, wk, acc, bias):
    return pl.pallas_call(
        _tc_final_body,
        grid=(N // ROWBLK,),
        in_specs=[_rowspec(), _rowspec(), _rowspec(), _wspec(), _rowspec(),
                  pl.BlockSpec((1, D), lambda i: (0, 0))],
        out_specs=_rowspec(),
        out_shape=jax.ShapeDtypeStruct((N, D), jnp.float32),
    )(s0, s1, dinv, wk, acc, bias)


def kernel(x, edge_index, edge_attr, batch, Ws, bias):
    del edge_attr, batch
    e = edge_index.shape[1]
    row = edge_index[0].astype(jnp.int32)
    col = edge_index[1].astype(jnp.int32)

    per_tile = -(-e // (NW * CHUNK)) * CHUNK
    n_chunks = per_tile // CHUNK
    epad = per_tile * NW - e
    rowp = jnp.concatenate([row, jnp.zeros((epad,), jnp.int32)])
    colp = jnp.concatenate([col, jnp.full((epad,), N, jnp.int32)])
    row3 = rowp.reshape(NW, n_chunks, CHUNK)
    col3 = colp.reshape(NW, n_chunks, CHUNK)

    zeros_d = jnp.zeros((NA, D), jnp.float32)
    ones_d = jnp.ones((CHUNK, D), jnp.float32)

    deg_parts = _make_sc_degree(n_chunks)(col3, ones_d, zeros_d)
    d0 = deg_parts[0, :N, :]
    d1 = deg_parts[1, :N, :]

    g, acc, dinv = _tc_prep(x, d0, d1, Ws[0])

    hop = _make_sc_hop(n_chunks)
    for k in range(1, KHOPS + 1):
        s = hop(g, row3, col3, zeros_d)
        s0 = s[0, :N, :]
        s1 = s[1, :N, :]
        if k < KHOPS:
            g, acc = _tc_hop(s0, s1, dinv, Ws[k], acc)
        else:
            out = _tc_final(s0, s1, dinv, Ws[k], acc, bias.reshape(1, D))
    return out
